# Initial kernel scaffold; baseline (speedup 1.0000x reference)
import jax
import jax.numpy as jnp
from jax.experimental import pallas as pl


def kernel(x, edges, node_split, conv_w, conv_b, W1, b1, W2, b2):
    raise NotImplementedError("write your pallas kernel here")


# speedup vs baseline: 81.5743x; 81.5743x over previous
import functools

import jax
import jax.numpy as jnp
from jax import lax
from jax.experimental import pallas as pl
from jax.experimental.pallas import tpu as pltpu
from jax.experimental.pallas import tpu_sc as plsc

NC = 2
NS = 16
LANES = 16

HIGH = lax.Precision.HIGHEST
JB = 1024


def _pad_idx(a, n_chunks, h, n_trash):
    pad = n_chunks * 128 - a.shape[-1]
    if pad:
        t = h + (jnp.arange(pad, dtype=jnp.int32) % n_trash)
        t = jnp.broadcast_to(t, a.shape[:-1] + (pad,))
        a = jnp.concatenate([a, t], axis=-1)
    return a.reshape(a.shape[:-1] + (n_chunks, 128))


def _sc_deg(dst_idx, hp, n_chunks):
    mesh = plsc.VectorSubcoreMesh(core_axis_name="c", subcore_axis_name="s", num_cores=NC, num_subcores=NS)
    rpt = hp // NS

    @functools.partial(
        pl.kernel,
        out_type=jax.ShapeDtypeStruct((NC, hp, LANES), jnp.float32),
        mesh=mesh,
        scratch_types=[
            pltpu.VMEM((n_chunks, 128), jnp.int32),
            pltpu.VMEM((128, LANES), jnp.float32),
            pltpu.VMEM((128, LANES), jnp.float32),
            pltpu.VMEM_SHARED((hp, LANES), jnp.float32),
        ],
    )
    def k(dst_hbm, out_hbm, idx_v, ones_v, zero_v, acc_sh):
        c = lax.axis_index("c")
        s = lax.axis_index("s")
        one16 = jnp.ones((LANES,), jnp.float32)
        z16 = jnp.zeros((LANES,), jnp.float32)
        for i in range(128):
            ones_v[i, :] = one16
            zero_v[i, :] = z16
        for i in range(rpt // 128):
            pltpu.sync_copy(zero_v, acc_sh.at[pl.ds(s * rpt + i * 128, 128)])
        plsc.subcore_barrier()
        pltpu.sync_copy(dst_hbm.at[c, s], idx_v)
        for i in range(n_chunks):
            pltpu.sync_copy(ones_v, acc_sh.at[idx_v.at[i]], add=True)
        plsc.subcore_barrier()
        pltpu.sync_copy(acc_sh.at[pl.ds(s * rpt, rpt)],
                        out_hbm.at[c, pl.ds(s * rpt, rpt)])

    return k(dst_idx)


def _sc_scatter(zflat, wp, src_h1, dst_h1, src_u, dst_u, hp, nch, nck_h1, nck_u):
    mesh = plsc.VectorSubcoreMesh(core_axis_name="c", subcore_axis_name="s", num_cores=NC, num_subcores=NS)
    rpt = hp // NS
    n_pass = nch // NC

    @functools.partial(
        pl.kernel,
        out_type=(jax.ShapeDtypeStruct((nch, hp, 128), jnp.float32),
                  jax.ShapeDtypeStruct((NC, hp, 128), jnp.float32)),
        mesh=mesh,
        scratch_types=[
            pltpu.VMEM((16, 128), jnp.int32),
            pltpu.VMEM((16, 128), jnp.int32),
            pltpu.VMEM((128, 128), jnp.float32),
            pltpu.VMEM((128, 128), jnp.float32),
            pltpu.VMEM((16, 128), jnp.float32),
            pltpu.VMEM_SHARED((hp, 128), jnp.float32),
            pltpu.SemaphoreType.DMA,
            pltpu.SemaphoreType.DMA,
        ],
    )
    def k(z_hbm, wp_hbm, srch_hbm, dsth_hbm, srcu_hbm, dstu_hbm,
          h1_hbm, u_hbm, gidx, sidx, buf0, buf1, zero_v, acc_sh, sem0, sem1):
        c = lax.axis_index("c")
        s = lax.axis_index("s")
        z16 = jnp.zeros((LANES,), jnp.float32)
        for i in range(16):
            for j in range(8):
                zero_v[i, pl.ds(j * 16, 16)] = z16
        bufs = (buf0, buf1)
        sems = (sem0, sem1)

        def zero_acc():
            for i in range(rpt // 16):
                pltpu.sync_copy(zero_v, acc_sh.at[pl.ds(s * rpt + i * 16, 16)])

        def scatter_pass(tbl_hbm, gsrc_hbm, ssrc_hbm, nck):
            for w0 in range(0, nck, 16):
                nw = min(16, nck - w0)
                pltpu.sync_copy(gsrc_hbm.at[pl.ds(w0, nw)], gidx.at[pl.ds(0, nw)])
                pltpu.sync_copy(ssrc_hbm.at[pl.ds(w0, nw)], sidx.at[pl.ds(0, nw)])
                pltpu.async_copy(tbl_hbm.at[gidx.at[0]], bufs[0], sems[0])
                for i in range(nw):
                    cur = bufs[i % 2]
                    pltpu.make_async_copy(tbl_hbm.at[gidx.at[i]], cur,
                                          sems[i % 2]).wait()
                    if i + 1 < nw:
                        pltpu.async_copy(tbl_hbm.at[gidx.at[i + 1]],
                                         bufs[(i + 1) % 2], sems[(i + 1) % 2])
                    pltpu.sync_copy(cur, acc_sh.at[sidx.at[i]], add=True)

        for p in range(n_pass):
            zero_acc()
            plsc.subcore_barrier()
            scatter_pass(z_hbm, srch_hbm.at[c, p, s], dsth_hbm.at[s], nck_h1)
            plsc.subcore_barrier()
            chunk = n_pass * c + p
            pltpu.sync_copy(acc_sh.at[pl.ds(s * rpt, rpt)],
                            h1_hbm.at[chunk, pl.ds(s * rpt, rpt)])
            plsc.subcore_barrier()

        zero_acc()
        plsc.subcore_barrier()
        scatter_pass(wp_hbm, dstu_hbm.at[c, s], srcu_hbm.at[c, s], nck_u)
        plsc.subcore_barrier()
        pltpu.sync_copy(acc_sh.at[pl.ds(s * rpt, rpt)],
                        u_hbm.at[c, pl.ds(s * rpt, rpt)])

    return k(zflat, wp, src_h1, dst_h1, src_u, dst_u)


def _tc1(x3, d0, d1, ns, cw2, cb, W1, hp, TP, P, D, G, GS):
    nchunk = TP * D // 128
    grid = hp // JB

    def body(x3_ref, d0_ref, d1_ref, ns_ref, cw2_ref, cb_ref, w1_ref,
             zp_ref, wp_ref, dv_ref):
        deg = d0_ref[:, 0:1] + d1_ref[:, 0:1] + 1.0
        dinv = lax.rsqrt(deg)
        dv_ref[...] = jnp.broadcast_to(dinv, (JB, LANES))
        A = lax.dot_general(cw2_ref[...], w1_ref[...],
                            (((0,), (0,)), ((), ())), precision=HIGH)
        cvec = lax.dot_general(cb_ref[...], w1_ref[...],
                               (((1,), (0,)), ((), ())), precision=HIGH)
        for tp in range(TP):
            xtp = x3_ref[tp]
            ztp = lax.dot_general(xtp, A, (((0,), (0,)), ((), ())),
                                  precision=HIGH)
            ztp = (ztp + cvec) * dinv
            col = tp * D
            zp_ref[col // 128, :, (col % 128):(col % 128) + D] = ztp
        jg = pl.program_id(0) * JB + lax.broadcasted_iota(jnp.int32, (JB, 1), 0)
        cols = []
        for gi in range(G):
            nsrow = ns_ref[gi:gi + 1, :]
            cnt = jnp.sum(jnp.where(jg == nsrow, 1.0, 0.0),
                          axis=1, keepdims=True)
            cols.append(dinv * cnt * (1.0 / GS))
        cols.append(jnp.zeros((JB, 128 - G), jnp.float32))
        wp_ref[...] = jnp.concatenate(cols, axis=1)

    return pl.pallas_call(
        body,
        grid=(grid,),
        in_specs=[
            pl.BlockSpec((TP, P, JB), lambda j: (0, 0, j)),
            pl.BlockSpec((JB, LANES), lambda j: (j, 0)),
            pl.BlockSpec((JB, LANES), lambda j: (j, 0)),
            pl.BlockSpec(ns.shape, lambda j: (0, 0)),
            pl.BlockSpec(cw2.shape, lambda j: (0, 0)),
            pl.BlockSpec(cb.shape, lambda j: (0, 0)),
            pl.BlockSpec(W1.shape, lambda j: (0, 0)),
        ],
        out_specs=[
            pl.BlockSpec((nchunk, JB, 128), lambda j: (0, j, 0)),
            pl.BlockSpec((JB, 128), lambda j: (j, 0)),
            pl.BlockSpec((JB, LANES), lambda j: (j, 0)),
        ],
        out_shape=[
            jax.ShapeDtypeStruct((nchunk, hp, 128), jnp.float32),
            jax.ShapeDtypeStruct((hp, 128), jnp.float32),
            jax.ShapeDtypeStruct((hp, LANES), jnp.float32),
        ],
    )(x3, d0, d1, ns, cw2, cb, W1)


def _tc2(zp, h1, u0, u1, wp, dv, W2, b1c, b2r, hp, TP, D, H):
    nchunk = TP * D // 128
    grid = hp // JB

    def body(zp_ref, h1_ref, u0_ref, u1_ref, wp_ref, dv_ref, w2_ref,
             b1_ref, b2_ref, out_ref, yacc):
        j = pl.program_id(0)

        @pl.when(j == 0)
        def _init():
            yacc[...] = jnp.zeros_like(yacc)

        jg = j * JB + lax.broadcasted_iota(jnp.int32, (JB, 1), 0)
        vmaskf = jnp.where(jg < H, 1.0, 0.0)
        dvb = dv_ref[...]
        dinv = dvb[:, 0:1]
        u2 = dvb * (u0_ref[:, :LANES] + u1_ref[:, :LANES]
                    + wp_ref[:, :LANES]) * vmaskf
        for k in range(nchunk):
            rk = jnp.maximum(dinv * (h1_ref[k] + zp_ref[k]) + b1_ref[...], 0.0)
            rk = rk * vmaskf
            yacc[:, k * 128:(k + 1) * 128] += lax.dot_general(
                u2, rk, (((0,), (0,)), ((), ())), precision=HIGH)

        @pl.when(j == grid - 1)
        def _head():
            for tp in range(TP):
                yt = yacc[:, tp * D:(tp + 1) * D]
                out_ref[tp] = lax.dot_general(
                    yt, w2_ref[...], (((1,), (0,)), ((), ())),
                    precision=HIGH) + b2_ref[...]

    return pl.pallas_call(
        body,
        grid=(grid,),
        in_specs=[
            pl.BlockSpec((nchunk, JB, 128), lambda j: (0, j, 0)),
            pl.BlockSpec((nchunk, JB, 128), lambda j: (0, j, 0)),
            pl.BlockSpec((JB, 128), lambda j: (j, 0)),
            pl.BlockSpec((JB, 128), lambda j: (j, 0)),
            pl.BlockSpec((JB, 128), lambda j: (j, 0)),
            pl.BlockSpec((JB, LANES), lambda j: (j, 0)),
            pl.BlockSpec(W2.shape, lambda j: (0, 0)),
            pl.BlockSpec((1, 128), lambda j: (0, 0)),
            pl.BlockSpec((1, D), lambda j: (0, 0)),
        ],
        out_specs=pl.BlockSpec((TP, LANES, D), lambda j: (0, 0, 0)),
        out_shape=jax.ShapeDtypeStruct((TP, LANES, D), jnp.float32),
        scratch_shapes=[pltpu.VMEM((LANES, TP * D), jnp.float32)],
        compiler_params=pltpu.CompilerParams(
            dimension_semantics=("arbitrary",)),
    )(zp, h1, u0, u1, wp, dv, W2, b1c, b2r)


def kernel(x, edges, node_split, conv_w, conv_b, W1, b1, W2, b2):
    N, _, T, H, _ = x.shape
    D, _, P = conv_w.shape
    TP = T // P
    G, GS = node_split.shape
    E = edges.shape[0]
    nchunk = TP * D // 128

    hp = -(-H // JB) * JB
    if hp == H:
        hp += JB
    ntrash = hp - H

    xs = x.reshape(T, H)
    x3 = jnp.pad(xs, ((0, 0), (0, hp - H))).reshape(TP, P, hp)

    src = edges[:, 0]
    dst = edges[:, 1]
    ep = -(-E // (NC * NS)) * (NC * NS)
    if ep != E:
        t = H + (jnp.arange(ep - E, dtype=jnp.int32) % ntrash)
        src = jnp.concatenate([src, t])
        dst = jnp.concatenate([dst, t])
    ew = ep // (NC * NS)
    eh = ep // NS
    nck_deg = -(-ew // 128)
    nck_h1 = -(-eh // 128)
    nck_u = nck_deg

    dst_deg = _pad_idx(dst.reshape(NC, NS, ew), nck_deg, H, ntrash)
    src_h1b = _pad_idx(src.reshape(NS, eh), nck_h1, H, ntrash)
    dst_h1 = _pad_idx(dst.reshape(NS, eh), nck_h1, H, ntrash)
    n_pass = nchunk // NC
    offs = (jnp.arange(NC)[:, None] * n_pass + jnp.arange(n_pass)[None, :]) * hp
    src_h1 = src_h1b[None, None] + offs[:, :, None, None, None].astype(jnp.int32)
    src_u = _pad_idx(src.reshape(NC, NS, ew), nck_u, H, ntrash)
    dst_u = _pad_idx(dst.reshape(NC, NS, ew), nck_u, H, ntrash)

    cw2 = conv_w[:, 0, :]
    cb = conv_b.reshape(1, D)
    b1c = jnp.tile(b1, 128 // D).reshape(1, 128)
    b2r = b2.reshape(1, D)

    degp = _sc_deg(dst_deg, hp, nck_deg)
    zp, wp, dv = _tc1(x3, degp[0], degp[1], node_split, cw2, cb, W1,
                      hp, TP, P, D, G, GS)
    zflat = zp.reshape(nchunk * hp, 128)
    h1raw, uraw = _sc_scatter(zflat, wp, src_h1, dst_h1, src_u, dst_u,
                              hp, nchunk, nck_h1, nck_u)
    out = _tc2(zp, h1raw, uraw[0], uraw[1], wp, dv, W2, b1c, b2r,
               hp, TP, D, H)
    return out[:, :G, :].reshape(N, TP * G, D)


# speedup vs baseline: 113.4239x; 1.3904x over previous
import functools

import jax
import jax.numpy as jnp
from jax import lax
from jax.experimental import pallas as pl
from jax.experimental.pallas import tpu as pltpu
from jax.experimental.pallas import tpu_sc as plsc

NC = 2
NS = 16
LANES = 16

HIGH = lax.Precision.HIGHEST
JB = 1024


def _pad_idx(a, n_chunks, h, n_trash, w=128):
    pad = n_chunks * w - a.shape[-1]
    if pad:
        t = h + (jnp.arange(pad, dtype=jnp.int32) % n_trash)
        t = jnp.broadcast_to(t, a.shape[:-1] + (pad,))
        a = jnp.concatenate([a, t], axis=-1)
    return a.reshape(a.shape[:-1] + (n_chunks, w))


def _sc_deg(dst_idx, hp, n_chunks):
    mesh = plsc.VectorSubcoreMesh(core_axis_name="c", subcore_axis_name="s",
                                  num_cores=NC, num_subcores=NS)
    rpt = hp // NS

    @functools.partial(
        pl.kernel,
        out_type=jax.ShapeDtypeStruct((NC, hp, LANES), jnp.float32),
        mesh=mesh,
        scratch_types=[
            pltpu.VMEM((n_chunks, 128), jnp.int32),
            pltpu.VMEM((128, LANES), jnp.float32),
            pltpu.VMEM((128, LANES), jnp.float32),
            pltpu.VMEM_SHARED((hp, LANES), jnp.float32),
        ],
    )
    def k(dst_hbm, out_hbm, idx_v, ones_v, zero_v, acc_sh):
        c = lax.axis_index("c")
        s = lax.axis_index("s")
        one16 = jnp.ones((LANES,), jnp.float32)
        z16 = jnp.zeros((LANES,), jnp.float32)
        for i in range(128):
            ones_v[i, :] = one16
            zero_v[i, :] = z16
        for i in range(rpt // 128):
            pltpu.sync_copy(zero_v, acc_sh.at[pl.ds(s * rpt + i * 128, 128)])
        plsc.subcore_barrier()
        pltpu.sync_copy(dst_hbm.at[c, s], idx_v)
        for i in range(n_chunks):
            pltpu.sync_copy(ones_v, acc_sh.at[idx_v.at[i]], add=True)
        plsc.subcore_barrier()
        pltpu.sync_copy(acc_sh.at[pl.ds(s * rpt, rpt)],
                        out_hbm.at[c, pl.ds(s * rpt, rpt)])

    return k(dst_idx)


def _sc_scatter(xp, wp, src_u, dst_u, hp, nck_u):
    mesh = plsc.VectorSubcoreMesh(core_axis_name="c", subcore_axis_name="s",
                                  num_cores=NC, num_subcores=NS)
    rpt = hp // NS

    @functools.partial(
        pl.kernel,
        out_type=(jax.ShapeDtypeStruct((NC, hp, 128), jnp.float32),
                  jax.ShapeDtypeStruct((NC, hp, 128), jnp.float32)),
        mesh=mesh,
        scratch_types=[
            pltpu.VMEM((32, 64), jnp.int32),
            pltpu.VMEM((32, 64), jnp.int32),
            pltpu.VMEM((64, 128), jnp.float32),
            pltpu.VMEM((64, 128), jnp.float32),
            pltpu.VMEM((64, 128), jnp.float32),
            pltpu.VMEM((64, 128), jnp.float32),
            pltpu.VMEM((16, 128), jnp.float32),
            pltpu.VMEM_SHARED((hp, 128), jnp.float32),
        ] + [pltpu.SemaphoreType.DMA] * 8,
    )
    def k(xp_hbm, wp_hbm, srcu_hbm, dstu_hbm,
          q_hbm, u_hbm, gidx, sidx, buf0, buf1, buf2, buf3, zero_v, acc_sh,
          gsem0, gsem1, gsem2, gsem3, ssem0, ssem1, ssem2, ssem3):
        c = lax.axis_index("c")
        s = lax.axis_index("s")
        z16 = jnp.zeros((LANES,), jnp.float32)
        for i in range(16):
            for j in range(8):
                zero_v[i, pl.ds(j * 16, 16)] = z16
        bufs = (buf0, buf1, buf2, buf3)
        gsems = (gsem0, gsem1, gsem2, gsem3)
        ssems = (ssem0, ssem1, ssem2, ssem3)
        NBUF = 4
        WIN = 32

        def zero_acc():
            for i in range(rpt // 16):
                pltpu.sync_copy(zero_v, acc_sh.at[pl.ds(s * rpt + i * 16, 16)])

        def scatter_pass(tbl_hbm, gsrc_hbm, ssrc_hbm, nck):
            for w0 in range(0, nck, WIN):
                nw = min(WIN, nck - w0)
                pltpu.sync_copy(gsrc_hbm.at[pl.ds(w0, nw)], gidx.at[pl.ds(0, nw)])
                pltpu.sync_copy(ssrc_hbm.at[pl.ds(w0, nw)], sidx.at[pl.ds(0, nw)])
                for j in range(min(NBUF - 1, nw)):
                    pltpu.async_copy(tbl_hbm.at[gidx.at[j]], bufs[j % NBUF],
                                     gsems[j % NBUF])
                for i in range(nw):
                    b = i % NBUF
                    pltpu.make_async_copy(tbl_hbm.at[gidx.at[i]], bufs[b],
                                          gsems[b]).wait()
                    pltpu.async_copy(bufs[b], acc_sh.at[sidx.at[i]], ssems[b],
                                     add=True)
                    j = i + NBUF - 1
                    if j < nw:
                        jb = j % NBUF
                        if j - NBUF >= 0:
                            pltpu.make_async_copy(bufs[jb],
                                                  acc_sh.at[sidx.at[j - NBUF]],
                                                  ssems[jb]).wait()
                        pltpu.async_copy(tbl_hbm.at[gidx.at[j]], bufs[jb],
                                         gsems[jb])
                for q in range(max(0, nw - NBUF), nw):
                    pltpu.make_async_copy(bufs[q % NBUF],
                                          acc_sh.at[sidx.at[q]],
                                          ssems[q % NBUF]).wait()

        zero_acc()
        plsc.subcore_barrier()
        scatter_pass(xp_hbm, srcu_hbm.at[c, s], dstu_hbm.at[c, s], nck_u)
        plsc.subcore_barrier()
        pltpu.sync_copy(acc_sh.at[pl.ds(s * rpt, rpt)],
                        q_hbm.at[c, pl.ds(s * rpt, rpt)])
        plsc.subcore_barrier()

        zero_acc()
        plsc.subcore_barrier()
        scatter_pass(wp_hbm, dstu_hbm.at[c, s], srcu_hbm.at[c, s], nck_u)
        plsc.subcore_barrier()
        pltpu.sync_copy(acc_sh.at[pl.ds(s * rpt, rpt)],
                        u_hbm.at[c, pl.ds(s * rpt, rpt)])

    return k(xp, wp, src_u, dst_u)


def _tc1(xst, d0, d1, ns, hp, T, G, GS):
    grid = hp // JB

    def body(xst_ref, d0_ref, d1_ref, ns_ref, xp_ref, wp_ref, dv_ref):
        deg = d0_ref[:, 0:1] + d1_ref[:, 0:1] + 1.0
        dinv = lax.rsqrt(deg)
        dv_ref[...] = jnp.broadcast_to(dinv, (JB, LANES))
        xp_ref[...] = jnp.concatenate(
            [dinv * xst_ref[...],
             jnp.zeros((JB, 127 - T), jnp.float32),
             dinv], axis=1)
        jg = pl.program_id(0) * JB + lax.broadcasted_iota(jnp.int32, (JB, 1), 0)
        cols = []
        for gi in range(G):
            nsrow = ns_ref[gi:gi + 1, :]
            cnt = jnp.sum(jnp.where(jg == nsrow, 1.0, 0.0),
                          axis=1, keepdims=True)
            cols.append(dinv * cnt * (1.0 / GS))
        cols.append(jnp.zeros((JB, 128 - G), jnp.float32))
        wp_ref[...] = jnp.concatenate(cols, axis=1)

    return pl.pallas_call(
        body,
        grid=(grid,),
        in_specs=[
            pl.BlockSpec((JB, T), lambda j: (j, 0)),
            pl.BlockSpec((JB, LANES), lambda j: (j, 0)),
            pl.BlockSpec((JB, LANES), lambda j: (j, 0)),
            pl.BlockSpec(ns.shape, lambda j: (0, 0)),
        ],
        out_specs=[
            pl.BlockSpec((JB, 128), lambda j: (j, 0)),
            pl.BlockSpec((JB, 128), lambda j: (j, 0)),
            pl.BlockSpec((JB, LANES), lambda j: (j, 0)),
        ],
        out_shape=[
            jax.ShapeDtypeStruct((hp, 128), jnp.float32),
            jax.ShapeDtypeStruct((hp, 128), jnp.float32),
            jax.ShapeDtypeStruct((hp, LANES), jnp.float32),
        ],
    )(xst, d0, d1, ns)


def _tc2(q0, q1, xp, u0, u1, wp, dv, cw2, cb, W1, W2, b1r, b2r,
         hp, TP, P, D, H):
    grid = hp // JB

    def body(q0_ref, q1_ref, xp_ref, u0_ref, u1_ref, wp_ref, dv_ref,
             cw2_ref, cb_ref, w1_ref, w2_ref, b1_ref, b2_ref, out_ref, yacc):
        j = pl.program_id(0)

        @pl.when(j == 0)
        def _init():
            yacc[...] = jnp.zeros_like(yacc)

        A = lax.dot_general(cw2_ref[...], w1_ref[...],
                            (((0,), (0,)), ((), ())), precision=HIGH)
        cvec = lax.dot_general(cb_ref[...], w1_ref[...],
                               (((1,), (0,)), ((), ())), precision=HIGH)
        jg = j * JB + lax.broadcasted_iota(jnp.int32, (JB, 1), 0)
        vmaskf = jnp.where(jg < H, 1.0, 0.0)
        dvb = dv_ref[...]
        dinv = dvb[:, 0:1]
        u2 = dvb * (u0_ref[:, :LANES] + u1_ref[:, :LANES]
                    + wp_ref[:, :LANES]) * vmaskf
        qf = q0_ref[...] + q1_ref[...] + xp_ref[...]
        qc = qf[:, 127:128] * cvec
        for tp in range(TP):
            ztp = lax.dot_general(qf[:, tp * P:(tp + 1) * P], A,
                                  (((1,), (0,)), ((), ())), precision=HIGH)
            rtp = jnp.maximum(dinv * (ztp + qc) + b1_ref[...], 0.0) * vmaskf
            yacc[:, tp * D:(tp + 1) * D] += lax.dot_general(
                u2, rtp, (((0,), (0,)), ((), ())), precision=HIGH)

        @pl.when(j == grid - 1)
        def _head():
            for tp in range(TP):
                yt = yacc[:, tp * D:(tp + 1) * D]
                out_ref[tp] = lax.dot_general(
                    yt, w2_ref[...], (((1,), (0,)), ((), ())),
                    precision=HIGH) + b2_ref[...]

    return pl.pallas_call(
        body,
        grid=(grid,),
        in_specs=[
            pl.BlockSpec((JB, 128), lambda j: (j, 0)),
            pl.BlockSpec((JB, 128), lambda j: (j, 0)),
            pl.BlockSpec((JB, 128), lambda j: (j, 0)),
            pl.BlockSpec((JB, 128), lambda j: (j, 0)),
            pl.BlockSpec((JB, 128), lambda j: (j, 0)),
            pl.BlockSpec((JB, 128), lambda j: (j, 0)),
            pl.BlockSpec((JB, LANES), lambda j: (j, 0)),
            pl.BlockSpec(cw2.shape, lambda j: (0, 0)),
            pl.BlockSpec(cb.shape, lambda j: (0, 0)),
            pl.BlockSpec(W1.shape, lambda j: (0, 0)),
            pl.BlockSpec(W2.shape, lambda j: (0, 0)),
            pl.BlockSpec((1, D), lambda j: (0, 0)),
            pl.BlockSpec((1, D), lambda j: (0, 0)),
        ],
        out_specs=pl.BlockSpec((TP, LANES, D), lambda j: (0, 0, 0)),
        out_shape=jax.ShapeDtypeStruct((TP, LANES, D), jnp.float32),
        scratch_shapes=[pltpu.VMEM((LANES, TP * D), jnp.float32)],
        compiler_params=pltpu.CompilerParams(
            dimension_semantics=("arbitrary",)),
    )(q0, q1, xp, u0, u1, wp, dv, cw2, cb, W1, W2, b1r, b2r)


def kernel(x, edges, node_split, conv_w, conv_b, W1, b1, W2, b2):
    N, _, T, H, _ = x.shape
    D, _, P = conv_w.shape
    TP = T // P
    G, GS = node_split.shape
    E = edges.shape[0]

    hp = -(-H // JB) * JB
    if hp == H:
        hp += JB
    ntrash = hp - H

    xs = x.reshape(T, H)
    xst = jnp.pad(xs.T, ((0, hp - H), (0, 0)))

    src = edges[:, 0]
    dst = edges[:, 1]
    ep = -(-E // (NC * NS)) * (NC * NS)
    if ep != E:
        t = H + (jnp.arange(ep - E, dtype=jnp.int32) % ntrash)
        src = jnp.concatenate([src, t])
        dst = jnp.concatenate([dst, t])
    ew = ep // (NC * NS)
    nck_deg = -(-ew // 128)
    nck_u = -(-ew // 64)

    dst_deg = _pad_idx(dst.reshape(NC, NS, ew), nck_deg, H, ntrash)
    src_u = _pad_idx(src.reshape(NC, NS, ew), nck_u, H, ntrash, 64)
    dst_u = _pad_idx(dst.reshape(NC, NS, ew), nck_u, H, ntrash, 64)

    cw2 = conv_w[:, 0, :]
    cb = conv_b.reshape(1, D)
    b1r = b1.reshape(1, D)
    b2r = b2.reshape(1, D)

    degp = _sc_deg(dst_deg, hp, nck_deg)
    xp, wp, dv = _tc1(xst, degp[0], degp[1], node_split, hp, T, G, GS)
    q, uraw = _sc_scatter(xp, wp, src_u, dst_u, hp, nck_u)
    out = _tc2(q[0], q[1], xp, uraw[0], uraw[1], wp, dv,
               cw2, cb, W1, W2, b1r, b2r, hp, TP, P, D, H)
    return out[:, :G, :].reshape(N, TP * G, D)
